# baseline (speedup 1.0000x reference)
import functools

import jax
import jax.numpy as jnp
from jax import lax
from jax.experimental import pallas as pl
from jax.experimental.pallas import tpu as pltpu
from jax.experimental.pallas import tpu_sc as plsc

VOCAB = 1000000
DIM = 64
DIMP = 128
B, L = 4096, 200
TOTAL = B * L

NC, NS = 2, 16
NW = NC * NS
PER_W = TOTAL // NW

CHUNK = 128
NBUF = 4
LAG = NBUF // 2
NCHUNK = PER_W // CHUNK
NGROUP = NCHUNK // NBUF

_mesh = plsc.VectorSubcoreMesh(core_axis_name="c", subcore_axis_name="s")


@functools.partial(
    pl.kernel,
    mesh=_mesh,
    out_type=jax.ShapeDtypeStruct((TOTAL, DIMP), jnp.float32),
    scratch_types=[
        pltpu.VMEM((PER_W,), jnp.int32),
    ]
    + [pltpu.VMEM((CHUNK, DIMP), jnp.float32) for _ in range(NBUF)]
    + [pltpu.SemaphoreType.DMA for _ in range(2 * NBUF)],
)
def _emb_lookup(tok_hbm, table_hbm, out_hbm, idx_v, *bufs_and_sems):
    bufs = bufs_and_sems[:NBUF]
    gsem = bufs_and_sems[NBUF : 2 * NBUF]
    wsem = bufs_and_sems[2 * NBUF : 3 * NBUF]

    wid = lax.axis_index("s") * NC + lax.axis_index("c")
    base = wid * PER_W

    pltpu.sync_copy(tok_hbm.at[pl.ds(base, PER_W)], idx_v)

    def gather(c, b):
        pltpu.async_copy(
            table_hbm.at[idx_v.at[pl.ds(c * CHUNK, CHUNK)]], bufs[b], gsem[b]
        )

    def write(c, b):
        pltpu.async_copy(
            bufs[b], out_hbm.at[pl.ds(base + c * CHUNK, CHUNK)], wsem[b]
        )

    def wait_gather(b):
        pltpu.make_async_copy(
            table_hbm.at[idx_v.at[pl.ds(0, CHUNK)]], bufs[b], gsem[b]
        ).wait()

    def wait_write(b):
        pltpu.make_async_copy(
            bufs[b], out_hbm.at[pl.ds(base, CHUNK)], wsem[b]
        ).wait()

    for b in range(LAG):
        gather(b, b)

    def group_body(g, carry):
        c0 = g * NBUF
        for b in range(NBUF):
            c = c0 + b
            tgt = (b + LAG) % NBUF
            wait_gather(b)
            write(c, b)
            if b < LAG:
                @pl.when(g == 0)
                def _():
                    gather(c + LAG, tgt)

                @pl.when(g > 0)
                def _():
                    wait_write(tgt)
                    gather(c + LAG, tgt)
            else:
                @pl.when(g < NGROUP - 1)
                def _():
                    wait_write(tgt)
                    gather(c + LAG, tgt)

        return carry

    lax.fori_loop(0, NGROUP, group_body, 0)

    for b in range(NBUF):
        wait_write(b)


TBLK = 512
TGRID = (VOCAB + TBLK - 1) // TBLK


def _transpose_pad_body(wt_ref, out_ref):
    out_ref[:, 0:DIM] = wt_ref[...].T


_transpose_pad = pl.pallas_call(
    _transpose_pad_body,
    grid=(TGRID,),
    in_specs=[pl.BlockSpec((DIM, TBLK), lambda j: (0, j))],
    out_specs=pl.BlockSpec((TBLK, DIMP), lambda j: (j, 0)),
    out_shape=jax.ShapeDtypeStruct((VOCAB, DIMP), jnp.float32),
)


def kernel(tokens, weight):
    tok_flat = tokens.reshape(TOTAL)
    wpad = _transpose_pad(weight.T)
    out = _emb_lookup(tok_flat, wpad)
    return out[:, :DIM].reshape(B, L, DIM)


# speedup vs baseline: 2.2346x; 2.2346x over previous
import functools

import jax
import jax.numpy as jnp
from jax import lax
from jax.experimental import pallas as pl
from jax.experimental.pallas import tpu as pltpu
from jax.experimental.pallas import tpu_sc as plsc

VOCAB = 1000000
DIM = 64
DIMP = 128
B, L = 4096, 200
TOTAL = B * L

NC, NS = 2, 16
NW = NC * NS
PER_W = TOTAL // NW

CHUNK = 128
NBUF = 4
LAG = NBUF // 2
NCHUNK = PER_W // CHUNK
NGROUP = NCHUNK // NBUF

_mesh = plsc.VectorSubcoreMesh(core_axis_name="c", subcore_axis_name="s")


@functools.partial(
    pl.kernel,
    mesh=_mesh,
    out_type=jax.ShapeDtypeStruct((TOTAL, DIMP), jnp.float32),
    scratch_types=[
        pltpu.VMEM((PER_W,), jnp.int32),
    ]
    + [pltpu.VMEM((CHUNK, DIMP), jnp.float32) for _ in range(NBUF)]
    + [pltpu.SemaphoreType.DMA for _ in range(2 * NBUF)],
)
def _emb_lookup(tok_hbm, table_hbm, out_hbm, idx_v, *bufs_and_sems):
    bufs = bufs_and_sems[:NBUF]
    gsem = bufs_and_sems[NBUF : 2 * NBUF]
    wsem = bufs_and_sems[2 * NBUF : 3 * NBUF]

    wid = lax.axis_index("s") * NC + lax.axis_index("c")
    base = wid * PER_W

    pltpu.sync_copy(tok_hbm.at[pl.ds(base, PER_W)], idx_v)

    def gather(c, b):
        pltpu.async_copy(
            table_hbm.at[idx_v.at[pl.ds(c * CHUNK, CHUNK)]], bufs[b], gsem[b]
        )

    def write(c, b):
        pltpu.async_copy(
            bufs[b], out_hbm.at[pl.ds(base + c * CHUNK, CHUNK)], wsem[b]
        )

    def wait_gather(b):
        pltpu.make_async_copy(
            table_hbm.at[idx_v.at[pl.ds(0, CHUNK)]], bufs[b], gsem[b]
        ).wait()

    def wait_write(b):
        pltpu.make_async_copy(
            bufs[b], out_hbm.at[pl.ds(base, CHUNK)], wsem[b]
        ).wait()

    for b in range(LAG):
        gather(b, b)

    def group_body(g, carry):
        c0 = g * NBUF
        for b in range(NBUF):
            c = c0 + b
            tgt = (b + LAG) % NBUF
            wait_gather(b)
            write(c, b)
            if b < LAG:
                @pl.when(g == 0)
                def _():
                    gather(c + LAG, tgt)

                @pl.when(g > 0)
                def _():
                    wait_write(tgt)
                    gather(c + LAG, tgt)
            else:
                @pl.when(g < NGROUP - 1)
                def _():
                    wait_write(tgt)
                    gather(c + LAG, tgt)

        return carry

    lax.fori_loop(0, NGROUP, group_body, 0)

    for b in range(NBUF):
        wait_write(b)


TBLK = 8192
TGRID = (VOCAB + TBLK - 1) // TBLK


def _transpose_pad_body(wt_ref, out_ref):
    out_ref[:, 0:DIM] = wt_ref[...].T


_transpose_pad = pl.pallas_call(
    _transpose_pad_body,
    grid=(TGRID,),
    in_specs=[pl.BlockSpec((DIM, TBLK), lambda j: (0, j))],
    out_specs=pl.BlockSpec((TBLK, DIMP), lambda j: (j, 0)),
    out_shape=jax.ShapeDtypeStruct((VOCAB, DIMP), jnp.float32),
)


def kernel(tokens, weight):
    tok_flat = tokens.reshape(TOTAL)
    wpad = _transpose_pad(weight.T)
    out = _emb_lookup(tok_flat, wpad)
    return out[:, :DIM].reshape(B, L, DIM)


# speedup vs baseline: 2.2870x; 1.0234x over previous
import functools

import jax
import jax.numpy as jnp
from jax import lax
from jax.experimental import pallas as pl
from jax.experimental.pallas import tpu as pltpu
from jax.experimental.pallas import tpu_sc as plsc

VOCAB = 1000000
DIM = 64
DIMP = 128
B, L = 4096, 200
TOTAL = B * L

NC, NS = 2, 16
NW = NC * NS
PER_W = TOTAL // NW

CHUNK = 128
NBUF = 4
LAG = NBUF // 2
NCHUNK = PER_W // CHUNK
NGROUP = NCHUNK // NBUF

_mesh = plsc.VectorSubcoreMesh(core_axis_name="c", subcore_axis_name="s")


@functools.partial(
    pl.kernel,
    mesh=_mesh,
    out_type=jax.ShapeDtypeStruct((TOTAL, DIMP), jnp.float32),
    scratch_types=[
        pltpu.VMEM((PER_W,), jnp.int32),
    ]
    + [pltpu.VMEM((CHUNK, DIMP), jnp.float32) for _ in range(NBUF)]
    + [pltpu.SemaphoreType.DMA for _ in range(2 * NBUF)],
)
def _emb_lookup(tok_hbm, table_hbm, out_hbm, idx_v, *bufs_and_sems):
    bufs = bufs_and_sems[:NBUF]
    gsem = bufs_and_sems[NBUF : 2 * NBUF]
    wsem = bufs_and_sems[2 * NBUF : 3 * NBUF]

    wid = lax.axis_index("s") * NC + lax.axis_index("c")
    base = wid * PER_W

    pltpu.sync_copy(tok_hbm.at[pl.ds(base, PER_W)], idx_v)

    def gather(c, b):
        pltpu.async_copy(
            table_hbm.at[idx_v.at[pl.ds(c * CHUNK, CHUNK)]], bufs[b], gsem[b]
        )

    def write(c, b):
        pltpu.async_copy(
            bufs[b], out_hbm.at[pl.ds(base + c * CHUNK, CHUNK)], wsem[b]
        )

    def wait_gather(b):
        pltpu.make_async_copy(
            table_hbm.at[idx_v.at[pl.ds(0, CHUNK)]], bufs[b], gsem[b]
        ).wait()

    def wait_write(b):
        pltpu.make_async_copy(
            bufs[b], out_hbm.at[pl.ds(base, CHUNK)], wsem[b]
        ).wait()

    for b in range(LAG):
        gather(b, b)

    def group_body(g, carry):
        c0 = g * NBUF
        for b in range(NBUF):
            c = c0 + b
            tgt = (b + LAG) % NBUF
            wait_gather(b)
            write(c, b)
            if b < LAG:
                @pl.when(g == 0)
                def _():
                    gather(c + LAG, tgt)

                @pl.when(g > 0)
                def _():
                    wait_write(tgt)
                    gather(c + LAG, tgt)
            else:
                @pl.when(g < NGROUP - 1)
                def _():
                    wait_write(tgt)
                    gather(c + LAG, tgt)

        return carry

    lax.fori_loop(0, NGROUP, group_body, 0)

    for b in range(NBUF):
        wait_write(b)


TBLK = 16384
TGRID = (VOCAB + TBLK - 1) // TBLK


def _transpose_pad_body(wt_ref, out_ref):
    out_ref[:, 0:DIM] = wt_ref[...].T


_transpose_pad = pl.pallas_call(
    _transpose_pad_body,
    grid=(TGRID,),
    in_specs=[pl.BlockSpec((DIM, TBLK), lambda j: (0, j))],
    out_specs=pl.BlockSpec((TBLK, DIMP), lambda j: (j, 0)),
    out_shape=jax.ShapeDtypeStruct((VOCAB, DIMP), jnp.float32),
)


def kernel(tokens, weight):
    tok_flat = tokens.reshape(TOTAL)
    wpad = _transpose_pad(weight.T)
    out = _emb_lookup(tok_flat, wpad)
    return out[:, :DIM].reshape(B, L, DIM)


# speedup vs baseline: 2.3150x; 1.0123x over previous
import functools

import jax
import jax.numpy as jnp
from jax import lax
from jax.experimental import pallas as pl
from jax.experimental.pallas import tpu as pltpu
from jax.experimental.pallas import tpu_sc as plsc

VOCAB = 1000000
DIM = 64
DIMP = 128
B, L = 4096, 200
TOTAL = B * L

NC, NS = 2, 16
NW = NC * NS
PER_W = TOTAL // NW

CHUNK = 64
NBUF = 8
LAG = NBUF // 2
NCHUNK = PER_W // CHUNK
NGROUP = NCHUNK // NBUF

_mesh = plsc.VectorSubcoreMesh(core_axis_name="c", subcore_axis_name="s")


@functools.partial(
    pl.kernel,
    mesh=_mesh,
    out_type=jax.ShapeDtypeStruct((TOTAL, DIMP), jnp.float32),
    scratch_types=[
        pltpu.VMEM((PER_W,), jnp.int32),
    ]
    + [pltpu.VMEM((CHUNK, DIMP), jnp.float32) for _ in range(NBUF)]
    + [pltpu.SemaphoreType.DMA for _ in range(2 * NBUF)],
)
def _emb_lookup(tok_hbm, table_hbm, out_hbm, idx_v, *bufs_and_sems):
    bufs = bufs_and_sems[:NBUF]
    gsem = bufs_and_sems[NBUF : 2 * NBUF]
    wsem = bufs_and_sems[2 * NBUF : 3 * NBUF]

    wid = lax.axis_index("s") * NC + lax.axis_index("c")
    base = wid * PER_W

    pltpu.sync_copy(tok_hbm.at[pl.ds(base, PER_W)], idx_v)

    def gather(c, b):
        pltpu.async_copy(
            table_hbm.at[idx_v.at[pl.ds(c * CHUNK, CHUNK)]], bufs[b], gsem[b]
        )

    def write(c, b):
        pltpu.async_copy(
            bufs[b], out_hbm.at[pl.ds(base + c * CHUNK, CHUNK)], wsem[b]
        )

    def wait_gather(b):
        pltpu.make_async_copy(
            table_hbm.at[idx_v.at[pl.ds(0, CHUNK)]], bufs[b], gsem[b]
        ).wait()

    def wait_write(b):
        pltpu.make_async_copy(
            bufs[b], out_hbm.at[pl.ds(base, CHUNK)], wsem[b]
        ).wait()

    for b in range(LAG):
        gather(b, b)

    def group_body(g, carry):
        c0 = g * NBUF
        for b in range(NBUF):
            c = c0 + b
            tgt = (b + LAG) % NBUF
            wait_gather(b)
            write(c, b)
            if b < LAG:
                @pl.when(g == 0)
                def _():
                    gather(c + LAG, tgt)

                @pl.when(g > 0)
                def _():
                    wait_write(tgt)
                    gather(c + LAG, tgt)
            else:
                @pl.when(g < NGROUP - 1)
                def _():
                    wait_write(tgt)
                    gather(c + LAG, tgt)

        return carry

    lax.fori_loop(0, NGROUP, group_body, 0)

    for b in range(NBUF):
        wait_write(b)


TBLK = 32768
TGRID = (VOCAB + TBLK - 1) // TBLK


def _transpose_pad_body(wt_ref, out_ref):
    out_ref[:, 0:DIM] = wt_ref[...].T


_transpose_pad = pl.pallas_call(
    _transpose_pad_body,
    grid=(TGRID,),
    in_specs=[pl.BlockSpec((DIM, TBLK), lambda j: (0, j))],
    out_specs=pl.BlockSpec((TBLK, DIMP), lambda j: (j, 0)),
    out_shape=jax.ShapeDtypeStruct((VOCAB, DIMP), jnp.float32),
)


def kernel(tokens, weight):
    tok_flat = tokens.reshape(TOTAL)
    wpad = _transpose_pad(weight.T)
    out = _emb_lookup(tok_flat, wpad)
    return out[:, :DIM].reshape(B, L, DIM)
